# Initial kernel scaffold; baseline (speedup 1.0000x reference)
import jax
import jax.numpy as jnp
from jax.experimental import pallas as pl


def kernel(node_logits, edge_logits, edge_index, node_factor_weights, edge_factor_weights):
    raise NotImplementedError("write your pallas kernel here")


# speedup vs baseline: 21.3315x; 21.3315x over previous
import functools

import jax
import jax.numpy as jnp
from jax import lax
from jax.experimental import pallas as pl
from jax.experimental.pallas import tpu as pltpu
from jax.experimental.pallas import tpu_sc as plsc

NUM_ITERATIONS = 2
GAMMA = 1.0

_NC = 2
_NS = 16
_NW = _NC * _NS
_L = 16

_EDGE_BLOCK = 2000
_EDGE_ROWS = 2048
_NODE_ROWS = 2048


def _npad(n_nodes):
    return ((n_nodes + _NW * 4 - 1) // (_NW * 4)) * (_NW * 4)


@functools.lru_cache(maxsize=None)
def _build_edge_pass(n_edges, n_nodes, with_degree):
    npad = _npad(n_nodes)
    epw = n_edges // _NW
    block = _EDGE_BLOCK
    nblocks = epw // block
    zchunk = npad // _NW
    zslice = npad // _NS
    del zchunk

    mesh = plsc.VectorSubcoreMesh(core_axis_name="c", subcore_axis_name="s",
                                  num_cores=_NC, num_subcores=_NS)

    out_type = [jax.ShapeDtypeStruct((n_edges,), jnp.float32),
                jax.ShapeDtypeStruct((_NC, npad), jnp.float32)]
    scratch = [pltpu.VMEM((n_nodes,), jnp.float32),
               pltpu.VMEM((block,), jnp.int32),
               pltpu.VMEM((block,), jnp.int32),
               pltpu.VMEM((block,), jnp.float32),
               pltpu.VMEM((zslice,), jnp.float32),
               pltpu.VMEM((_L,), jnp.float32),
               pltpu.VMEM_SHARED((npad,), jnp.float32)]
    if with_degree:
        out_type.append(jax.ShapeDtypeStruct((_NC, npad), jnp.float32))
        scratch.append(pltpu.VMEM((block,), jnp.float32))
        scratch.append(pltpu.VMEM_SHARED((npad,), jnp.float32))

    def body(src_hbm, dst_hbm, sn_hbm, se_hbm, gm_hbm,
             snew_hbm, sums_hbm, *rest):
        if with_degree:
            deg_hbm = rest[0]
            (table_v, src_v, dst_v, s_v, zero_v, gm_v, sums_sh,
             ones_v, deg_sh) = rest[1:]
        else:
            (table_v, src_v, dst_v, s_v, zero_v, gm_v, sums_sh) = rest

        cid = lax.axis_index("c")
        sid = lax.axis_index("s")
        wid = cid * _NS + sid

        pltpu.sync_copy(sn_hbm, table_v)
        pltpu.sync_copy(gm_hbm, gm_v)

        def zstep(i, carry):
            zero_v[pl.ds(i * _L, _L)] = jnp.zeros((_L,), jnp.float32)
            return carry
        lax.fori_loop(0, zslice // _L, zstep, 0)
        pltpu.sync_copy(zero_v, sums_sh.at[pl.ds(sid * zslice, zslice)])
        if with_degree:
            pltpu.sync_copy(zero_v, deg_sh.at[pl.ds(sid * zslice, zslice)])

            def ostep(i, carry):
                ones_v[pl.ds(i * _L, _L)] = jnp.ones((_L,), jnp.float32)
                return carry
            lax.fori_loop(0, block // _L, ostep, 0)
        plsc.subcore_barrier()

        base0 = wid * epw

        def do_block(b, carry):
            base = base0 + b * block
            pltpu.sync_copy(src_hbm.at[pl.ds(base, block)], src_v)
            pltpu.sync_copy(dst_hbm.at[pl.ds(base, block)], dst_v)
            pltpu.sync_copy(se_hbm.at[pl.ds(base, block)], s_v)
            gm = gm_v[...]

            def step(i, c2):
                o = i * _L
                isrc = src_v[pl.ds(o, _L)]
                idst = dst_v[pl.ds(o, _L)]
                nap = jnp.maximum(plsc.load_gather(table_v, [isrc]),
                                  plsc.load_gather(table_v, [idst]))
                f = 1.0 + nap * gm
                s = s_v[pl.ds(o, _L)]
                sf = s * f
                s_v[pl.ds(o, _L)] = sf / (1.0 - s + sf)
                return c2
            lax.fori_loop(0, block // _L, step, 0)

            pltpu.sync_copy(s_v, snew_hbm.at[pl.ds(base, block)])
            pltpu.sync_copy(s_v, sums_sh.at[src_v], add=True)
            pltpu.sync_copy(s_v, sums_sh.at[dst_v], add=True)
            if with_degree:
                pltpu.sync_copy(ones_v, deg_sh.at[src_v], add=True)
                pltpu.sync_copy(ones_v, deg_sh.at[dst_v], add=True)
            return carry
        lax.fori_loop(0, nblocks, do_block, 0)

        plsc.subcore_barrier()

        @pl.when(sid == 0)
        def _():
            pltpu.sync_copy(sums_sh, sums_hbm.at[cid])
            if with_degree:
                pltpu.sync_copy(deg_sh, deg_hbm.at[cid])

    return pl.kernel(body, out_type=tuple(out_type), mesh=mesh,
                     scratch_types=tuple(scratch),
                     compiler_params=pltpu.CompilerParams(
                         needs_layout_passes=False))


def _abnormal_body(x_ref, o_ref):
    x = x_ref[...]
    m = jnp.max(x, axis=1, keepdims=True)
    e = jnp.exp(x - m)
    z = jnp.sum(e, axis=1)
    o_ref[...] = 1.0 - e[:, 0] / z


@functools.lru_cache(maxsize=None)
def _build_abnormal(rows, n_classes, block_rows):
    grid = pl.cdiv(rows, block_rows)
    return pl.pallas_call(
        _abnormal_body,
        grid=(grid,),
        in_specs=[pl.BlockSpec((block_rows, n_classes), lambda i: (i, 0))],
        out_specs=pl.BlockSpec((block_rows,), lambda i: (i,)),
        out_shape=jax.ShapeDtypeStruct((rows,), jnp.float32),
    )


@functools.lru_cache(maxsize=None)
def _build_node_update(n_nodes, first):
    npad = _npad(n_nodes)

    def body(s_ref, sums_ref, deg_ref, gm_ref, snew_ref, *rest):
        s = s_ref[...]
        sums = sums_ref[0, :n_nodes] + sums_ref[1, :n_nodes]
        if first:
            deg = deg_ref[0, :n_nodes] + deg_ref[1, :n_nodes]
        else:
            deg = deg_ref[...]
        m = sums / (deg + 1e-6)
        f = 1.0 + gm_ref[0] * m
        sf = s * f
        snew_ref[...] = sf / (1.0 - s + sf)
        if first:
            rest[0][...] = deg

    deg_spec = (pl.BlockSpec((_NC, npad), lambda: (0, 0)) if first
                else pl.BlockSpec((n_nodes,), lambda: (0,)))
    if first:
        out_shape = (jax.ShapeDtypeStruct((n_nodes,), jnp.float32),
                     jax.ShapeDtypeStruct((n_nodes,), jnp.float32))
        out_specs = (pl.BlockSpec((n_nodes,), lambda: (0,)),
                     pl.BlockSpec((n_nodes,), lambda: (0,)))
    else:
        out_shape = jax.ShapeDtypeStruct((n_nodes,), jnp.float32)
        out_specs = pl.BlockSpec((n_nodes,), lambda: (0,))

    return pl.pallas_call(
        body,
        grid=(),
        in_specs=[pl.BlockSpec((n_nodes,), lambda: (0,)),
                  pl.BlockSpec((_NC, npad), lambda: (0, 0)),
                  deg_spec,
                  pl.BlockSpec(memory_space=pltpu.SMEM)],
        out_specs=out_specs,
        out_shape=out_shape,
    )


def _refine_body(n_classes, x_ref, s0_ref, s2_ref, o_ref):
    x = x_ref[...]
    m = jnp.max(x, axis=1, keepdims=True)
    e = jnp.exp(x - m)
    z = jnp.sum(e, axis=1, keepdims=True)
    p = e / z
    s0 = s0_ref[...]
    s2 = s2_ref[...]
    r = (s2 / jnp.maximum(s0, 1e-30))[:, None]
    col = lax.broadcasted_iota(jnp.int32, x.shape, 1)
    vals = jnp.where(col == 0, (1.0 - s2)[:, None], p * r)
    o_ref[...] = jnp.log(vals + 1e-9)


@functools.lru_cache(maxsize=None)
def _build_refine(rows, n_classes, block_rows):
    grid = pl.cdiv(rows, block_rows)
    return pl.pallas_call(
        functools.partial(_refine_body, n_classes),
        grid=(grid,),
        in_specs=[pl.BlockSpec((block_rows, n_classes), lambda i: (i, 0)),
                  pl.BlockSpec((block_rows,), lambda i: (i,)),
                  pl.BlockSpec((block_rows,), lambda i: (i,))],
        out_specs=pl.BlockSpec((block_rows, n_classes), lambda i: (i, 0)),
        out_shape=jax.ShapeDtypeStruct((rows, n_classes), jnp.float32),
    )


def kernel(node_logits, edge_logits, edge_index, node_factor_weights,
           edge_factor_weights):
    n_nodes, node_classes = node_logits.shape
    n_edges, edge_classes = edge_logits.shape

    src = edge_index[0].astype(jnp.int32)
    dst = edge_index[1].astype(jnp.int32)

    gm_e = GAMMA * jnp.mean(edge_factor_weights[1:, 1:].astype(jnp.float32))
    gm_n = GAMMA * jnp.mean(node_factor_weights[1:, 1:].astype(jnp.float32))
    gm_e_vec = jnp.full((_L,), gm_e, jnp.float32)
    gm_n_s = jnp.reshape(gm_n, (1,))

    s_e = _build_abnormal(n_edges, edge_classes, _EDGE_ROWS)(edge_logits)
    s_n = _build_abnormal(n_nodes, node_classes, _NODE_ROWS)(node_logits)
    s_e0, s_n0 = s_e, s_n

    edge_pass_deg = _build_edge_pass(n_edges, n_nodes, True)
    edge_pass = _build_edge_pass(n_edges, n_nodes, False)
    node_upd1 = _build_node_update(n_nodes, True)
    node_upd2 = _build_node_update(n_nodes, False)

    deg = None
    for it in range(NUM_ITERATIONS):
        if it == 0:
            s_e, sums, deg_p = edge_pass_deg(src, dst, s_n, s_e, gm_e_vec)
            s_n, deg = node_upd1(s_n, sums, deg_p, gm_n_s)
        else:
            s_e, sums = edge_pass(src, dst, s_n, s_e, gm_e_vec)
            s_n = node_upd2(s_n, sums, deg, gm_n_s)

    node_out = _build_refine(n_nodes, node_classes, _NODE_ROWS)(
        node_logits, s_n0, s_n)
    edge_out = _build_refine(n_edges, edge_classes, _EDGE_ROWS)(
        edge_logits, s_e0, s_e)
    return (node_out, edge_out)


# speedup vs baseline: 31.4124x; 1.4726x over previous
import functools

import jax
import jax.numpy as jnp
from jax import lax
from jax.experimental import pallas as pl
from jax.experimental.pallas import tpu as pltpu
from jax.experimental.pallas import tpu_sc as plsc

NUM_ITERATIONS = 2
GAMMA = 1.0

_NC = 2
_NS = 16
_NW = _NC * _NS
_L = 16

_EDGE_BLOCK = 2000
_EDGE_ROWS = 2048
_NODE_ROWS = 2048


def _npad(n_nodes):
    return ((n_nodes + _NW * 4 - 1) // (_NW * 4)) * (_NW * 4)


@functools.lru_cache(maxsize=None)
def _build_edge_pass(n_edges, n_nodes, with_degree):
    npad = _npad(n_nodes)
    epw = n_edges // _NW
    block = _EDGE_BLOCK
    nblocks = epw // block
    zchunk = npad // _NW
    zslice = npad // _NS
    del zchunk

    mesh = plsc.VectorSubcoreMesh(core_axis_name="c", subcore_axis_name="s",
                                  num_cores=_NC, num_subcores=_NS)

    out_type = [jax.ShapeDtypeStruct((n_edges,), jnp.float32),
                jax.ShapeDtypeStruct((_NC, npad), jnp.float32)]
    scratch = [pltpu.VMEM((n_nodes,), jnp.float32),
               pltpu.VMEM((block,), jnp.int32),
               pltpu.VMEM((block,), jnp.int32),
               pltpu.VMEM((block,), jnp.float32),
               pltpu.VMEM((zslice,), jnp.float32),
               pltpu.VMEM((_L,), jnp.float32),
               pltpu.VMEM_SHARED((npad,), jnp.float32)]
    if with_degree:
        out_type.append(jax.ShapeDtypeStruct((_NC, npad), jnp.float32))
        scratch.append(pltpu.VMEM((block,), jnp.float32))
        scratch.append(pltpu.VMEM_SHARED((npad,), jnp.float32))

    def body(src_hbm, dst_hbm, sn_hbm, se_hbm, gm_hbm,
             snew_hbm, sums_hbm, *rest):
        if with_degree:
            deg_hbm = rest[0]
            (table_v, src_v, dst_v, s_v, zero_v, gm_v, sums_sh,
             ones_v, deg_sh) = rest[1:]
        else:
            (table_v, src_v, dst_v, s_v, zero_v, gm_v, sums_sh) = rest

        cid = lax.axis_index("c")
        sid = lax.axis_index("s")
        wid = cid * _NS + sid

        pltpu.sync_copy(sn_hbm, table_v)
        pltpu.sync_copy(gm_hbm, gm_v)

        def zstep(i, carry):
            zero_v[pl.ds(i * _L, _L)] = jnp.zeros((_L,), jnp.float32)
            return carry
        lax.fori_loop(0, zslice // _L, zstep, 0)
        pltpu.sync_copy(zero_v, sums_sh.at[pl.ds(sid * zslice, zslice)])
        if with_degree:
            pltpu.sync_copy(zero_v, deg_sh.at[pl.ds(sid * zslice, zslice)])

            def ostep(i, carry):
                ones_v[pl.ds(i * _L, _L)] = jnp.ones((_L,), jnp.float32)
                return carry
            lax.fori_loop(0, block // _L, ostep, 0)
        plsc.subcore_barrier()

        base0 = wid * epw

        def do_block(b, carry):
            base = base0 + b * block
            pltpu.sync_copy(src_hbm.at[pl.ds(base, block)], src_v)
            pltpu.sync_copy(dst_hbm.at[pl.ds(base, block)], dst_v)
            pltpu.sync_copy(se_hbm.at[pl.ds(base, block)], s_v)
            gm = gm_v[...]

            def step(i, c2):
                o = i * _L
                isrc = src_v[pl.ds(o, _L)]
                idst = dst_v[pl.ds(o, _L)]
                nap = jnp.maximum(plsc.load_gather(table_v, [isrc]),
                                  plsc.load_gather(table_v, [idst]))
                f = 1.0 + nap * gm
                s = s_v[pl.ds(o, _L)]
                sf = s * f
                s_v[pl.ds(o, _L)] = sf / (1.0 - s + sf)
                return c2
            lax.fori_loop(0, block // _L, step, 0)

            pltpu.sync_copy(s_v, snew_hbm.at[pl.ds(base, block)])
            pltpu.sync_copy(s_v, sums_sh.at[src_v], add=True)
            pltpu.sync_copy(s_v, sums_sh.at[dst_v], add=True)
            if with_degree:
                pltpu.sync_copy(ones_v, deg_sh.at[src_v], add=True)
                pltpu.sync_copy(ones_v, deg_sh.at[dst_v], add=True)
            return carry
        lax.fori_loop(0, nblocks, do_block, 0)

        plsc.subcore_barrier()

        @pl.when(sid == 0)
        def _():
            pltpu.sync_copy(sums_sh, sums_hbm.at[cid])
            if with_degree:
                pltpu.sync_copy(deg_sh, deg_hbm.at[cid])

    return pl.kernel(body, out_type=tuple(out_type), mesh=mesh,
                     scratch_types=tuple(scratch),
                     compiler_params=pltpu.CompilerParams(
                         needs_layout_passes=False))


_ROW_BLOCK = 8000


def _gather_classes(buf_v, w, n_classes):
    ii = lax.iota(jnp.int32, _L) * n_classes + w * (_L * n_classes)
    return [plsc.load_gather(buf_v, [ii + c]) for c in range(n_classes)]


@functools.lru_cache(maxsize=None)
def _build_sc_abnormal(rows, n_classes):
    rpw = rows // _NW
    block = _ROW_BLOCK
    nblocks = rpw // block
    mesh = plsc.VectorSubcoreMesh(core_axis_name="c", subcore_axis_name="s",
                                  num_cores=_NC, num_subcores=_NS)

    def body(x_hbm, s_hbm, x_v, s_v):
        wid = lax.axis_index("c") * _NS + lax.axis_index("s")
        base0 = wid * rpw

        def do_block(b, carry):
            base = base0 + b * block
            pltpu.sync_copy(x_hbm.at[pl.ds(base * n_classes,
                                           block * n_classes)], x_v)

            def step(w, c2):
                v = _gather_classes(x_v, w, n_classes)
                m = v[0]
                for c in range(1, n_classes):
                    m = jnp.maximum(m, v[c])
                e0 = jnp.exp(v[0] - m)
                z = e0
                for c in range(1, n_classes):
                    z = z + jnp.exp(v[c] - m)
                s_v[pl.ds(w * _L, _L)] = 1.0 - e0 / z
                return c2
            lax.fori_loop(0, block // _L, step, 0)
            pltpu.sync_copy(s_v, s_hbm.at[pl.ds(base, block)])
            return carry
        lax.fori_loop(0, nblocks, do_block, 0)

    return pl.kernel(
        body,
        out_type=jax.ShapeDtypeStruct((rows,), jnp.float32),
        mesh=mesh,
        scratch_types=(pltpu.VMEM((block * n_classes,), jnp.float32),
                       pltpu.VMEM((block,), jnp.float32)),
        compiler_params=pltpu.CompilerParams(needs_layout_passes=False))


@functools.lru_cache(maxsize=None)
def _build_sc_refine(rows, n_classes):
    rpw = rows // _NW
    block = _ROW_BLOCK
    nblocks = rpw // block
    mesh = plsc.VectorSubcoreMesh(core_axis_name="c", subcore_axis_name="s",
                                  num_cores=_NC, num_subcores=_NS)

    def body(x_hbm, s0_hbm, s2_hbm, o_hbm, x_v, s0_v, s2_v, o_v):
        wid = lax.axis_index("c") * _NS + lax.axis_index("s")
        base0 = wid * rpw

        def do_block(b, carry):
            base = base0 + b * block
            pltpu.sync_copy(x_hbm.at[pl.ds(base * n_classes,
                                           block * n_classes)], x_v)
            pltpu.sync_copy(s0_hbm.at[pl.ds(base, block)], s0_v)
            pltpu.sync_copy(s2_hbm.at[pl.ds(base, block)], s2_v)

            def step(w, c2):
                ii = (lax.iota(jnp.int32, _L) * n_classes
                      + w * (_L * n_classes))
                v = [plsc.load_gather(x_v, [ii + c])
                     for c in range(n_classes)]
                m = v[0]
                for c in range(1, n_classes):
                    m = jnp.maximum(m, v[c])
                e = [jnp.exp(vc - m) for vc in v]
                z = e[0]
                for c in range(1, n_classes):
                    z = z + e[c]
                s0 = s0_v[pl.ds(w * _L, _L)]
                s2 = s2_v[pl.ds(w * _L, _L)]
                r = s2 / jnp.maximum(s0, 1e-30)
                plsc.store_scatter(o_v, [ii], 1.0 - s2)
                for c in range(1, n_classes):
                    plsc.store_scatter(o_v, [ii + c], e[c] / z * r)
                return c2
            lax.fori_loop(0, block // _L, step, 0)
            pltpu.sync_copy(o_v, o_hbm.at[pl.ds(base * n_classes,
                                                block * n_classes)])
            return carry
        lax.fori_loop(0, nblocks, do_block, 0)

    return pl.kernel(
        body,
        out_type=jax.ShapeDtypeStruct((rows * n_classes,), jnp.float32),
        mesh=mesh,
        scratch_types=(pltpu.VMEM((block * n_classes,), jnp.float32),
                       pltpu.VMEM((block,), jnp.float32),
                       pltpu.VMEM((block,), jnp.float32),
                       pltpu.VMEM((block * n_classes,), jnp.float32)),
        compiler_params=pltpu.CompilerParams(needs_layout_passes=False))


_LOG_COLS = 1280
_LOG_ROWS = 1000


def _log_body(x_ref, o_ref):
    o_ref[...] = jnp.log(x_ref[...] + 1e-9)


@functools.lru_cache(maxsize=None)
def _build_log(rows, cols, block_rows):
    grid = rows // block_rows
    return pl.pallas_call(
        _log_body,
        grid=(grid,),
        in_specs=[pl.BlockSpec((block_rows, cols), lambda i: (i, 0))],
        out_specs=pl.BlockSpec((block_rows, cols), lambda i: (i, 0)),
        out_shape=jax.ShapeDtypeStruct((rows, cols), jnp.float32),
    )


def _abnormal_body(x_ref, o_ref):
    x = x_ref[...]
    m = jnp.max(x, axis=1, keepdims=True)
    e = jnp.exp(x - m)
    z = jnp.sum(e, axis=1)
    o_ref[...] = 1.0 - e[:, 0] / z


@functools.lru_cache(maxsize=None)
def _build_abnormal(rows, n_classes, block_rows):
    grid = pl.cdiv(rows, block_rows)
    return pl.pallas_call(
        _abnormal_body,
        grid=(grid,),
        in_specs=[pl.BlockSpec((block_rows, n_classes), lambda i: (i, 0))],
        out_specs=pl.BlockSpec((block_rows,), lambda i: (i,)),
        out_shape=jax.ShapeDtypeStruct((rows,), jnp.float32),
    )


@functools.lru_cache(maxsize=None)
def _build_node_update(n_nodes, first):
    npad = _npad(n_nodes)

    def body(s_ref, sums_ref, deg_ref, gm_ref, snew_ref, *rest):
        s = s_ref[...]
        sums = sums_ref[0, :n_nodes] + sums_ref[1, :n_nodes]
        if first:
            deg = deg_ref[0, :n_nodes] + deg_ref[1, :n_nodes]
        else:
            deg = deg_ref[...]
        m = sums / (deg + 1e-6)
        f = 1.0 + gm_ref[0] * m
        sf = s * f
        snew_ref[...] = sf / (1.0 - s + sf)
        if first:
            rest[0][...] = deg

    deg_spec = (pl.BlockSpec((_NC, npad), lambda: (0, 0)) if first
                else pl.BlockSpec((n_nodes,), lambda: (0,)))
    if first:
        out_shape = (jax.ShapeDtypeStruct((n_nodes,), jnp.float32),
                     jax.ShapeDtypeStruct((n_nodes,), jnp.float32))
        out_specs = (pl.BlockSpec((n_nodes,), lambda: (0,)),
                     pl.BlockSpec((n_nodes,), lambda: (0,)))
    else:
        out_shape = jax.ShapeDtypeStruct((n_nodes,), jnp.float32)
        out_specs = pl.BlockSpec((n_nodes,), lambda: (0,))

    return pl.pallas_call(
        body,
        grid=(),
        in_specs=[pl.BlockSpec((n_nodes,), lambda: (0,)),
                  pl.BlockSpec((_NC, npad), lambda: (0, 0)),
                  deg_spec,
                  pl.BlockSpec(memory_space=pltpu.SMEM)],
        out_specs=out_specs,
        out_shape=out_shape,
    )


def _refine_body(n_classes, x_ref, s0_ref, s2_ref, o_ref):
    x = x_ref[...]
    m = jnp.max(x, axis=1, keepdims=True)
    e = jnp.exp(x - m)
    z = jnp.sum(e, axis=1, keepdims=True)
    p = e / z
    s0 = s0_ref[...]
    s2 = s2_ref[...]
    r = (s2 / jnp.maximum(s0, 1e-30))[:, None]
    col = lax.broadcasted_iota(jnp.int32, x.shape, 1)
    vals = jnp.where(col == 0, (1.0 - s2)[:, None], p * r)
    o_ref[...] = jnp.log(vals + 1e-9)


@functools.lru_cache(maxsize=None)
def _build_refine(rows, n_classes, block_rows):
    grid = pl.cdiv(rows, block_rows)
    return pl.pallas_call(
        functools.partial(_refine_body, n_classes),
        grid=(grid,),
        in_specs=[pl.BlockSpec((block_rows, n_classes), lambda i: (i, 0)),
                  pl.BlockSpec((block_rows,), lambda i: (i,)),
                  pl.BlockSpec((block_rows,), lambda i: (i,))],
        out_specs=pl.BlockSpec((block_rows, n_classes), lambda i: (i, 0)),
        out_shape=jax.ShapeDtypeStruct((rows, n_classes), jnp.float32),
    )


def kernel(node_logits, edge_logits, edge_index, node_factor_weights,
           edge_factor_weights):
    n_nodes, node_classes = node_logits.shape
    n_edges, edge_classes = edge_logits.shape

    src = edge_index[0].astype(jnp.int32)
    dst = edge_index[1].astype(jnp.int32)

    gm_e = GAMMA * jnp.mean(edge_factor_weights[1:, 1:].astype(jnp.float32))
    gm_n = GAMMA * jnp.mean(node_factor_weights[1:, 1:].astype(jnp.float32))
    gm_e_vec = jnp.full((_L,), gm_e, jnp.float32)
    gm_n_s = jnp.reshape(gm_n, (1,))

    edge_flat = jnp.reshape(edge_logits, (-1,))
    s_e = _build_sc_abnormal(n_edges, edge_classes)(edge_flat)
    s_n = _build_abnormal(n_nodes, node_classes, _NODE_ROWS)(node_logits)
    s_e0, s_n0 = s_e, s_n

    edge_pass_deg = _build_edge_pass(n_edges, n_nodes, True)
    edge_pass = _build_edge_pass(n_edges, n_nodes, False)
    node_upd1 = _build_node_update(n_nodes, True)
    node_upd2 = _build_node_update(n_nodes, False)

    deg = None
    for it in range(NUM_ITERATIONS):
        if it == 0:
            s_e, sums, deg_p = edge_pass_deg(src, dst, s_n, s_e, gm_e_vec)
            s_n, deg = node_upd1(s_n, sums, deg_p, gm_n_s)
        else:
            s_e, sums = edge_pass(src, dst, s_n, s_e, gm_e_vec)
            s_n = node_upd2(s_n, sums, deg, gm_n_s)

    node_out = _build_refine(n_nodes, node_classes, _NODE_ROWS)(
        node_logits, s_n0, s_n)
    vals_flat = _build_sc_refine(n_edges, edge_classes)(edge_flat, s_e0, s_e)
    total = n_edges * edge_classes
    log_rows = total // _LOG_COLS
    vals2d = jnp.reshape(vals_flat, (log_rows, _LOG_COLS))
    edge_out = jnp.reshape(
        _build_log(log_rows, _LOG_COLS, _LOG_ROWS)(vals2d),
        (n_edges, edge_classes))
    return (node_out, edge_out)


# speedup vs baseline: 162.3313x; 5.1677x over previous
import functools

import jax
import jax.numpy as jnp
from jax import lax
from jax.experimental import pallas as pl
from jax.experimental.pallas import tpu as pltpu
from jax.experimental.pallas import tpu_sc as plsc

NUM_ITERATIONS = 2
GAMMA = 1.0

_NC = 2
_NS = 16
_NW = _NC * _NS
_L = 16

_EDGE_BLOCK = 2000


def _npad(n_nodes):
    return ((n_nodes + _NW * 4 - 1) // (_NW * 4)) * (_NW * 4)


@functools.lru_cache(maxsize=None)
def _build_edge_pass(n_edges, n_nodes, with_degree):
    npad = _npad(n_nodes)
    epw = n_edges // _NW
    block = _EDGE_BLOCK
    nblocks = epw // block
    zchunk = npad // _NW
    zslice = npad // _NS
    del zchunk

    mesh = plsc.VectorSubcoreMesh(core_axis_name="c", subcore_axis_name="s",
                                  num_cores=_NC, num_subcores=_NS)

    out_type = [jax.ShapeDtypeStruct((n_edges,), jnp.float32),
                jax.ShapeDtypeStruct((_NC, npad), jnp.float32)]
    scratch = [pltpu.VMEM((n_nodes,), jnp.float32),
               pltpu.VMEM((block,), jnp.int32),
               pltpu.VMEM((block,), jnp.int32),
               pltpu.VMEM((block,), jnp.float32),
               pltpu.VMEM((zslice,), jnp.float32),
               pltpu.VMEM((_L,), jnp.float32),
               pltpu.VMEM_SHARED((npad,), jnp.float32)]
    if with_degree:
        out_type.append(jax.ShapeDtypeStruct((_NC, npad), jnp.float32))
        scratch.append(pltpu.VMEM((block,), jnp.float32))
        scratch.append(pltpu.VMEM_SHARED((npad,), jnp.float32))

    def body(src_hbm, dst_hbm, sn_hbm, se_hbm, gm_hbm,
             snew_hbm, sums_hbm, *rest):
        if with_degree:
            deg_hbm = rest[0]
            (table_v, src_v, dst_v, s_v, zero_v, gm_v, sums_sh,
             ones_v, deg_sh) = rest[1:]
        else:
            (table_v, src_v, dst_v, s_v, zero_v, gm_v, sums_sh) = rest

        cid = lax.axis_index("c")
        sid = lax.axis_index("s")
        wid = cid * _NS + sid

        pltpu.sync_copy(sn_hbm, table_v)
        pltpu.sync_copy(gm_hbm, gm_v)

        def zstep(i, carry):
            zero_v[pl.ds(i * _L, _L)] = jnp.zeros((_L,), jnp.float32)
            return carry
        lax.fori_loop(0, zslice // _L, zstep, 0)
        pltpu.sync_copy(zero_v, sums_sh.at[pl.ds(sid * zslice, zslice)])
        if with_degree:
            pltpu.sync_copy(zero_v, deg_sh.at[pl.ds(sid * zslice, zslice)])

            def ostep(i, carry):
                ones_v[pl.ds(i * _L, _L)] = jnp.ones((_L,), jnp.float32)
                return carry
            lax.fori_loop(0, block // _L, ostep, 0)
        plsc.subcore_barrier()

        base0 = wid * epw

        def do_block(b, carry):
            base = base0 + b * block
            pltpu.sync_copy(src_hbm.at[pl.ds(base, block)], src_v)
            pltpu.sync_copy(dst_hbm.at[pl.ds(base, block)], dst_v)
            pltpu.sync_copy(se_hbm.at[pl.ds(base, block)], s_v)
            gm = gm_v[...]

            def step(i, c2):
                o = i * _L
                isrc = src_v[pl.ds(o, _L)]
                idst = dst_v[pl.ds(o, _L)]
                nap = jnp.maximum(plsc.load_gather(table_v, [isrc]),
                                  plsc.load_gather(table_v, [idst]))
                f = 1.0 + nap * gm
                s = s_v[pl.ds(o, _L)]
                sf = s * f
                s_v[pl.ds(o, _L)] = sf / (1.0 - s + sf)
                return c2
            lax.fori_loop(0, block // _L, step, 0)

            pltpu.sync_copy(s_v, snew_hbm.at[pl.ds(base, block)])
            pltpu.sync_copy(s_v, sums_sh.at[src_v], add=True)
            pltpu.sync_copy(s_v, sums_sh.at[dst_v], add=True)
            if with_degree:
                pltpu.sync_copy(ones_v, deg_sh.at[src_v], add=True)
                pltpu.sync_copy(ones_v, deg_sh.at[dst_v], add=True)
            return carry
        lax.fori_loop(0, nblocks, do_block, 0)

        plsc.subcore_barrier()

        @pl.when(sid == 0)
        def _():
            pltpu.sync_copy(sums_sh, sums_hbm.at[cid])
            if with_degree:
                pltpu.sync_copy(deg_sh, deg_hbm.at[cid])

    return pl.kernel(body, out_type=tuple(out_type), mesh=mesh,
                     scratch_types=tuple(scratch),
                     compiler_params=pltpu.CompilerParams(
                         needs_layout_passes=False))


_EDGE_COLS = 51200
_NODE_COLS = 16384


def _abnormal_t_body(x_ref, o_ref):
    x = x_ref[...]
    m = jnp.max(x, axis=0)
    e = jnp.exp(x - m[None, :])
    z = jnp.sum(e, axis=0)
    o_ref[...] = 1.0 - e[0, :] / z


@functools.lru_cache(maxsize=None)
def _build_abnormal(cols, n_classes, block_cols):
    grid = pl.cdiv(cols, block_cols)
    return pl.pallas_call(
        _abnormal_t_body,
        grid=(grid,),
        in_specs=[pl.BlockSpec((n_classes, block_cols), lambda i: (0, i))],
        out_specs=pl.BlockSpec((block_cols,), lambda i: (i,)),
        out_shape=jax.ShapeDtypeStruct((cols,), jnp.float32),
    )


@functools.lru_cache(maxsize=None)
def _build_node_update(n_nodes, first):
    npad = _npad(n_nodes)

    def body(s_ref, sums_ref, deg_ref, gm_ref, snew_ref, *rest):
        s = s_ref[...]
        sums = sums_ref[0, :n_nodes] + sums_ref[1, :n_nodes]
        if first:
            deg = deg_ref[0, :n_nodes] + deg_ref[1, :n_nodes]
        else:
            deg = deg_ref[...]
        m = sums / (deg + 1e-6)
        f = 1.0 + gm_ref[0] * m
        sf = s * f
        snew_ref[...] = sf / (1.0 - s + sf)
        if first:
            rest[0][...] = deg

    deg_spec = (pl.BlockSpec((_NC, npad), lambda: (0, 0)) if first
                else pl.BlockSpec((n_nodes,), lambda: (0,)))
    if first:
        out_shape = (jax.ShapeDtypeStruct((n_nodes,), jnp.float32),
                     jax.ShapeDtypeStruct((n_nodes,), jnp.float32))
        out_specs = (pl.BlockSpec((n_nodes,), lambda: (0,)),
                     pl.BlockSpec((n_nodes,), lambda: (0,)))
    else:
        out_shape = jax.ShapeDtypeStruct((n_nodes,), jnp.float32)
        out_specs = pl.BlockSpec((n_nodes,), lambda: (0,))

    return pl.pallas_call(
        body,
        grid=(),
        in_specs=[pl.BlockSpec((n_nodes,), lambda: (0,)),
                  pl.BlockSpec((_NC, npad), lambda: (0, 0)),
                  deg_spec,
                  pl.BlockSpec(memory_space=pltpu.SMEM)],
        out_specs=out_specs,
        out_shape=out_shape,
    )


def _refine_t_body(x_ref, s0_ref, s2_ref, o_ref):
    x = x_ref[...]
    m = jnp.max(x, axis=0, keepdims=True)
    e = jnp.exp(x - m)
    z = jnp.sum(e, axis=0, keepdims=True)
    p = e / z
    s0 = s0_ref[...]
    s2 = s2_ref[...]
    r = (s2 / jnp.maximum(s0, 1e-30))[None, :]
    row = lax.broadcasted_iota(jnp.int32, x.shape, 0)
    vals = jnp.where(row == 0, (1.0 - s2)[None, :], p * r)
    o_ref[...] = jnp.log(vals + 1e-9)


@functools.lru_cache(maxsize=None)
def _build_refine(cols, n_classes, block_cols):
    grid = pl.cdiv(cols, block_cols)
    return pl.pallas_call(
        _refine_t_body,
        grid=(grid,),
        in_specs=[pl.BlockSpec((n_classes, block_cols), lambda i: (0, i)),
                  pl.BlockSpec((block_cols,), lambda i: (i,)),
                  pl.BlockSpec((block_cols,), lambda i: (i,))],
        out_specs=pl.BlockSpec((n_classes, block_cols), lambda i: (0, i)),
        out_shape=jax.ShapeDtypeStruct((n_classes, cols), jnp.float32),
    )


def kernel(node_logits, edge_logits, edge_index, node_factor_weights,
           edge_factor_weights):
    n_nodes, node_classes = node_logits.shape
    n_edges, edge_classes = edge_logits.shape

    src = edge_index[0].astype(jnp.int32)
    dst = edge_index[1].astype(jnp.int32)

    gm_e = GAMMA * jnp.mean(edge_factor_weights[1:, 1:].astype(jnp.float32))
    gm_n = GAMMA * jnp.mean(node_factor_weights[1:, 1:].astype(jnp.float32))
    gm_e_vec = jnp.full((_L,), gm_e, jnp.float32)
    gm_n_s = jnp.reshape(gm_n, (1,))

    edge_t = jnp.transpose(edge_logits)
    node_t = jnp.transpose(node_logits)
    s_e = _build_abnormal(n_edges, edge_classes, _EDGE_COLS)(edge_t)
    s_n = _build_abnormal(n_nodes, node_classes, _NODE_COLS)(node_t)
    s_e0, s_n0 = s_e, s_n

    edge_pass_deg = _build_edge_pass(n_edges, n_nodes, True)
    edge_pass = _build_edge_pass(n_edges, n_nodes, False)
    node_upd1 = _build_node_update(n_nodes, True)
    node_upd2 = _build_node_update(n_nodes, False)

    deg = None
    for it in range(NUM_ITERATIONS):
        if it == 0:
            s_e, sums, deg_p = edge_pass_deg(src, dst, s_n, s_e, gm_e_vec)
            s_n, deg = node_upd1(s_n, sums, deg_p, gm_n_s)
        else:
            s_e, sums = edge_pass(src, dst, s_n, s_e, gm_e_vec)
            s_n = node_upd2(s_n, sums, deg, gm_n_s)

    node_out = jnp.transpose(
        _build_refine(n_nodes, node_classes, _NODE_COLS)(node_t, s_n0, s_n))
    edge_out = jnp.transpose(
        _build_refine(n_edges, edge_classes, _EDGE_COLS)(edge_t, s_e0, s_e))
    return (node_out, edge_out)
